# baseline (speedup 1.0000x reference)
import functools

import jax
import jax.numpy as jnp
from jax import lax
from jax.experimental import pallas as pl
from jax.experimental.pallas import tpu as pltpu
from jax.experimental.pallas import tpu_sc as plsc

T = 8192
D_MODEL = 2048
N_EXPERTS = 16
TILE_T = 1024

NC = 2
NS = 16
NW = NC * NS
TPW = T // NW


def _ground_body(alpha_ref, hidden_ref, logits_ref, eg_ref, out_ref):
    sim = jax.lax.dot_general(
        hidden_ref[...], eg_ref[...], (((1,), (1,)), ((), ())),
        preferred_element_type=jnp.float32,
    )
    out_ref[...] = logits_ref[...] + alpha_ref[0, 0] * sim


def _ground(token_hidden, router_logits, expert_ground, alpha):
    alpha_arr = jnp.reshape(alpha.astype(jnp.float32), (1, 1))
    return pl.pallas_call(
        _ground_body,
        grid=(T // TILE_T,),
        in_specs=[
            pl.BlockSpec(memory_space=pltpu.SMEM),
            pl.BlockSpec((TILE_T, D_MODEL), lambda i: (i, 0)),
            pl.BlockSpec((TILE_T, N_EXPERTS), lambda i: (i, 0)),
            pl.BlockSpec((N_EXPERTS, D_MODEL), lambda i: (0, 0)),
        ],
        out_specs=pl.BlockSpec((TILE_T, N_EXPERTS), lambda i: (i, 0)),
        out_shape=jax.ShapeDtypeStruct((T, N_EXPERTS), jnp.float32),
        compiler_params=pltpu.CompilerParams(
            dimension_semantics=("parallel",),
        ),
    )(alpha_arr, token_hidden, router_logits, expert_ground)


def _take16(x, idx):
    return lax.gather(
        x,
        idx[:, None],
        dimension_numbers=lax.GatherDimensionNumbers(
            offset_dims=(), collapsed_slice_dims=(0,), start_index_map=(0,)
        ),
        slice_sizes=(1,),
        mode=lax.GatherScatterMode.PROMISE_IN_BOUNDS,
    )


@functools.partial(
    pl.kernel,
    mesh=plsc.VectorSubcoreMesh(core_axis_name="c", subcore_axis_name="s"),
    out_type=jax.ShapeDtypeStruct((T, 16), jnp.float32),
    scratch_types=[
        pltpu.VMEM((TPW, N_EXPERTS), jnp.float32),
        pltpu.VMEM((TPW, 16), jnp.float32),
    ],
)
def _sc_top2(grounded_hbm, out_hbm, g_v, out_v):
    wid = lax.axis_index("s") * NC + lax.axis_index("c")
    base = wid * TPW
    pltpu.sync_copy(grounded_hbm.at[pl.ds(base, TPW), :], g_v)

    lane = lax.iota(jnp.int32, 16)
    slot = lane % 4
    quad = lane // 4
    neg_inf = jnp.float32(-jnp.inf)

    def bfly(x, op):
        for s in (8, 4, 2, 1):
            x = op(x, _take16(x, lane ^ s))
        return x

    def pack_one(row):
        m1 = bfly(row, jnp.maximum)
        i1 = bfly(jnp.where(row == m1, lane, N_EXPERTS), jnp.minimum)
        masked = jnp.where(lane == i1, neg_inf, row)
        m2 = bfly(masked, jnp.maximum)
        i2 = bfly(jnp.where(masked == m2, lane, N_EXPERTS), jnp.minimum)
        e = jnp.exp(m2 - m1)
        r = 1.0 / (1.0 + e)
        return jnp.where(
            slot == 0,
            i1.astype(jnp.float32),
            jnp.where(
                slot == 1,
                r,
                jnp.where(slot == 2, i2.astype(jnp.float32), e * r),
            ),
        )

    def one_token(t, carry):
        out_v[t] = pack_one(g_v[t])
        return carry

    lax.fori_loop(0, TPW, one_token, None, unroll=8)

    pltpu.sync_copy(out_v, out_hbm.at[pl.ds(base, TPW), :])


@jax.jit
def _run(token_hidden, router_logits, expert_ground, alpha):
    grounded = _ground(token_hidden, router_logits, expert_ground, alpha)
    packed = _sc_top2(grounded)
    return packed[:, :4].reshape(T, 2, 2)


def kernel(token_hidden, router_logits, expert_ground, alpha):
    return _run(token_hidden, router_logits, expert_ground, alpha)


# speedup vs baseline: 1.9451x; 1.9451x over previous
import jax
import jax.numpy as jnp
from jax.experimental import pallas as pl
from jax.experimental.pallas import tpu as pltpu

T = 8192
D_MODEL = 2048
N_EXPERTS = 16
TILE_T = 1024


def _ident(n):
    r = jax.lax.broadcasted_iota(jnp.int32, (n, n), 0)
    c = jax.lax.broadcasted_iota(jnp.int32, (n, n), 1)
    return (r == c).astype(jnp.float32)


def _routing_body(alpha_ref, hidden_ref, logits_ref, eg_ref, out_ref):
    alpha = alpha_ref[0, 0]
    sim = jax.lax.dot_general(
        hidden_ref[...], eg_ref[...], (((1,), (1,)), ((), ())),
        preferred_element_type=jnp.float32,
    )
    grounded = logits_ref[...] + alpha * sim

    g = jax.lax.transpose(grounded, (1, 0))

    idx = jax.lax.broadcasted_iota(jnp.int32, g.shape, 0)
    neg_inf = jnp.float32(-jnp.inf)

    m1 = jnp.max(g, axis=0, keepdims=True)
    i1 = jnp.min(jnp.where(g == m1, idx, N_EXPERTS), axis=0, keepdims=True)
    g2 = jnp.where(idx == i1, neg_inf, g)
    m2 = jnp.max(g2, axis=0, keepdims=True)
    i2 = jnp.min(jnp.where(g2 == m2, idx, N_EXPERTS), axis=0, keepdims=True)

    e = jnp.exp(m2 - m1)
    r = 1.0 / (1.0 + e)
    w1 = r
    w2 = e * r

    packed_t = jnp.concatenate(
        [i1.astype(jnp.float32), w1, i2.astype(jnp.float32), w2], axis=0
    )
    out_ref[...] = jax.lax.transpose(packed_t, (1, 0))


@jax.jit
def _run(token_hidden, router_logits, expert_ground, alpha):
    alpha_arr = jnp.reshape(alpha.astype(jnp.float32), (1, 1))
    packed = pl.pallas_call(
        _routing_body,
        grid=(T // TILE_T,),
        in_specs=[
            pl.BlockSpec(memory_space=pltpu.SMEM),
            pl.BlockSpec((TILE_T, D_MODEL), lambda i: (i, 0)),
            pl.BlockSpec((TILE_T, N_EXPERTS), lambda i: (i, 0)),
            pl.BlockSpec((N_EXPERTS, D_MODEL), lambda i: (0, 0)),
        ],
        out_specs=pl.BlockSpec((TILE_T, 4), lambda i: (i, 0)),
        out_shape=jax.ShapeDtypeStruct((T, 4), jnp.float32),
        compiler_params=pltpu.CompilerParams(
            dimension_semantics=("parallel",),
        ),
    )(alpha_arr, token_hidden, router_logits, expert_ground)
    return packed.reshape(T, 2, 2)


def kernel(token_hidden, router_logits, expert_ground, alpha):
    return _run(token_hidden, router_logits, expert_ground, alpha)
